# baseline (speedup 1.0000x reference)
import functools

import jax
import jax.numpy as jnp
from jax import lax
from jax.experimental import pallas as pl
from jax.experimental.pallas import tpu as pltpu
from jax.experimental.pallas import tpu_sc as plsc

BATCH = 16384
DIM = 64
NUM_CORES = 2
NUM_SUBCORES = 16
NUM_WORKERS = NUM_CORES * NUM_SUBCORES
B_PER_W = BATCH // NUM_WORKERS
CHUNK = 128
NCHUNK = B_PER_W // CHUNK


def _emb_body(idx_hbm, table_hbm, out_hbm, idx_v, rows_v, sem):
    wid = lax.axis_index("s") * NUM_CORES + lax.axis_index("c")
    base = wid * B_PER_W
    pltpu.sync_copy(idx_hbm.at[wid], idx_v)
    copies = [
        pltpu.async_copy(
            table_hbm.at[idx_v.at[j]],
            rows_v.at[pl.ds(j * CHUNK, CHUNK)],
            sem,
        )
        for j in range(NCHUNK)
    ]
    for c in copies:
        c.wait()
    pltpu.sync_copy(rows_v, out_hbm.at[pl.ds(base, B_PER_W)])


@jax.jit
def _embed(idx_grouped, weight):
    mesh = plsc.VectorSubcoreMesh(core_axis_name="c", subcore_axis_name="s")
    return pl.kernel(
        _emb_body,
        mesh=mesh,
        out_type=jax.ShapeDtypeStruct((BATCH, DIM), jnp.float32),
        scratch_types=[
            pltpu.VMEM((NCHUNK, CHUNK), jnp.int32),
            pltpu.VMEM((B_PER_W, DIM), jnp.float32),
            pltpu.SemaphoreType.DMA,
        ],
        compiler_params=pltpu.CompilerParams(use_tc_tiling_on_sc=False),
    )(idx_grouped, weight)


def kernel(indices, weight):
    idx_grouped = indices.astype(jnp.int32).reshape(NUM_WORKERS, NCHUNK, CHUNK)
    return _embed(idx_grouped, weight)


# speedup vs baseline: 3.2991x; 3.2991x over previous
import functools

import jax
import jax.numpy as jnp
from jax import lax
from jax.experimental import pallas as pl
from jax.experimental.pallas import tpu as pltpu
from jax.experimental.pallas import tpu_sc as plsc

BATCH = 16384
DIM = 64
NUM_TBL = 1000000
PHYS_COLS = 1000064
NUM_CORES = 2
NUM_SUBCORES = 16
NUM_WORKERS = NUM_CORES * NUM_SUBCORES
R_COLS = 31360
BLK = 512
NBLK = R_COLS // BLK + 1
LAST_SO = PHYS_COLS - BLK
HITCAP = BATCH + 32
DUMP_ROW = BATCH
STAGE = 16
SENTINEL = 2**30


def _emb_body(idx_hbm, wt_hbm, out_hbm, idx_v, blkbuf, hits, pend,
              rowstage, bidx, sems):
    wid = lax.axis_index("s") * NUM_CORES + lax.axis_index("c")
    w_lo = wid * R_COLS
    w_hi = jnp.minimum(w_lo + R_COLS, NUM_TBL)
    lane = lax.iota(jnp.int32, 16)

    pltpu.sync_copy(idx_hbm, idx_v)

    def scan_body(v, n):
        hv = idx_v[pl.ds(v * 16, 16)]
        m = jnp.logical_and(hv >= w_lo, hv < w_hi)
        cnt = plsc.all_reduce_population_count(m)[0]

        @pl.when(cnt > 0)
        def _():
            packed = (hv - w_lo) | ((v * 16 + lane) << 15)
            plsc.store_compressed(hits.at[pl.ds(n, 16)], packed, mask=m)

        return n + cnt

    n_hits = lax.fori_loop(0, BATCH // 16, scan_body, 0)
    hits[pl.ds(n_hits, 16)] = jnp.full((16,), SENTINEL, jnp.int32)
    n_hvregs = (n_hits + 15) >> 4

    bidx[...] = jnp.full((16,), jnp.int32(DUMP_ROW))

    def fetch(g, buf):
        s = w_lo + g * BLK

        @pl.when(s < NUM_TBL)
        def _():
            so = pl.multiple_of(jnp.minimum(s, LAST_SO), 128)
            pltpu.async_copy(
                wt_hbm.at[:, pl.ds(so, BLK)], blkbuf.at[buf], sems.at[buf]
            )

    def wait_fetch(g, buf):
        s = w_lo + g * BLK

        @pl.when(s < NUM_TBL)
        def _():
            pltpu.make_async_copy(
                wt_hbm.at[:, pl.ds(0, BLK)], blkbuf.at[buf], sems.at[buf]
            ).wait()

    def scalar_at(ref, k):
        return plsc.load_gather(ref, [jnp.full((16,), k, jnp.int32)])[0]

    dvecs = [16 * q + lane for q in range(4)]

    def flush():
        pltpu.sync_copy(rowstage, out_hbm.at[bidx])
        bidx[...] = jnp.full((16,), jnp.int32(DUMP_ROW))

    def block_body(g, slot):
        buf = lax.rem(g, 2)

        @pl.when(g + 1 < NBLK)
        def _():
            fetch(g + 1, 1 - buf)

        wait_fetch(g, buf)
        s = w_lo + g * BLK
        so = jnp.minimum(s, LAST_SO)
        rel_so = so - w_lo

        def match_body(v, npend):
            hp = hits[pl.ds(v * 16, 16)]
            m = (jnp.bitwise_and(hp, 0x7FFF) >> 9) == g
            cnt = plsc.all_reduce_population_count(m)[0]

            @pl.when(cnt > 0)
            def _():
                plsc.store_compressed(pend.at[pl.ds(npend, 16)], hp, mask=m)

            return npend + cnt

        npend = lax.fori_loop(0, n_hvregs, match_body, 0)

        def ext_cond(carry):
            k, _ = carry
            return k < npend

        def ext_body(carry):
            k, slot = carry
            hp = scalar_at(pend, k)
            l = jnp.bitwise_and(hp, 0x7FFF) - rel_so
            b = hp >> 15
            lv = jnp.full((16,), l, jnp.int32)
            bufv = jnp.full((16,), buf, jnp.int32)
            for q in range(4):
                g16 = plsc.load_gather(blkbuf, [bufv, dvecs[q], lv])
                rowstage[slot, pl.ds(16 * q, 16)] = g16
            plsc.store_scatter(bidx, [jnp.full((16,), slot, jnp.int32)],
                               jnp.full((16,), b, jnp.int32),
                               mask=lane == 0)
            slot = slot + 1

            @pl.when(slot == STAGE)
            def _():
                flush()

            return k + 1, lax.rem(slot, STAGE)

        _, slot = lax.while_loop(ext_cond, ext_body, (0, slot))
        return slot

    fetch(0, 0)
    slot = lax.fori_loop(0, NBLK, block_body, 0)

    @pl.when(slot > 0)
    def _():
        flush()


@jax.jit
def _embed(indices, weight):
    mesh = plsc.VectorSubcoreMesh(core_axis_name="c", subcore_axis_name="s")
    out_pad = pl.kernel(
        _emb_body,
        mesh=mesh,
        out_type=jax.ShapeDtypeStruct((BATCH + 128, 128), jnp.float32),
        scratch_types=[
            pltpu.VMEM((BATCH,), jnp.int32),
            pltpu.VMEM((2, DIM, BLK), jnp.float32),
            pltpu.VMEM((HITCAP,), jnp.int32),
            pltpu.VMEM((HITCAP,), jnp.int32),
            pltpu.VMEM((STAGE, 128), jnp.float32),
            pltpu.VMEM((16,), jnp.int32),
            pltpu.SemaphoreType.DMA((2,)),
        ],
        compiler_params=pltpu.CompilerParams(
            use_tc_tiling_on_sc=True, needs_layout_passes=False
        ),
    )(indices, weight.T)
    return out_pad[:BATCH, :DIM]


def kernel(indices, weight):
    return _embed(indices.astype(jnp.int32), weight)
